# baseline (speedup 1.0000x reference)
import dataclasses
import functools

import jax
import jax.numpy as jnp
from jax import lax
from jax.experimental import pallas as pl
from jax.experimental.pallas import tpu as pltpu
from jax.experimental.pallas import tpu_sc as plsc

N = 10000
NP = 10240
E = 160000
EP = 163840
FC = 128
NCORES = 2
NSUB = 16
BATCH = 128
TRASH = NP - 8
PAD_SRC = N
BM = 1280
ROWS_PER_SUB = NP // NSUB
DEG_NB = EP // (NCORES * NSUB) // BATCH
AGG_NB = EP // NSUB // BATCH
AGG_NBH = AGG_NB // 2

_mesh = plsc.VectorSubcoreMesh(core_axis_name="c", subcore_axis_name="s")

_sc_cp = pltpu.CompilerParams()
if "needs_layout_passes" in pltpu.CompilerParams.__dataclass_fields__:
    _sc_cp = dataclasses.replace(_sc_cp, needs_layout_passes=False)


HC = 8


def _sc_deg_body(dst_hbm, degp_hbm, didx, hist):
    c = lax.axis_index("c")
    s = lax.axis_index("s")
    wid = c * NSUB + s
    pltpu.sync_copy(dst_hbm.at[pl.ds(wid * DEG_NB, DEG_NB)], didx)

    @pl.loop(0, NP * HC, step=16)
    def _(i):
        hist[pl.ds(i, 16)] = jnp.zeros((16,), jnp.float32)

    ones_v = jnp.ones((16,), jnp.float32)

    @pl.loop(0, DEG_NB)
    def _(b):
        @pl.loop(0, BATCH, step=16)
        def _(j):
            idx = didx[b, pl.ds(j, 16)] * HC
            plsc.addupdate_scatter(hist, [idx], ones_v)

    pltpu.sync_copy(hist, degp_hbm.at[wid])


def _sc_deg(dst2d):
    return pl.kernel(
        _sc_deg_body,
        out_type=jax.ShapeDtypeStruct((NCORES * NSUB, NP * HC), jnp.float32),
        mesh=_mesh,
        compiler_params=_sc_cp,
        scratch_types=[
            pltpu.VMEM((DEG_NB, BATCH), jnp.int32),
            pltpu.VMEM((NP * HC,), jnp.float32),
        ],
    )(dst2d)


def _sc_agg_body(nch, ytab_hbm, src_hbm, dst_hbm, zeros_hbm, agg_hbm,
                 sidx, didx, gbuf, accum):
    c = lax.axis_index("c")
    s = lax.axis_index("s")
    pltpu.sync_copy(src_hbm.at[pl.ds(s * AGG_NB, AGG_NB)], sidx)
    pltpu.sync_copy(dst_hbm.at[pl.ds(s * AGG_NB, AGG_NB)], didx)
    for k in range(nch // NCORES):
        ch = c + NCORES * k
        pltpu.sync_copy(zeros_hbm,
                        accum.at[pl.ds(s * ROWS_PER_SUB, ROWS_PER_SUB)])
        plsc.subcore_barrier()

        @pl.loop(0, AGG_NB)
        def _(b):
            pltpu.sync_copy(ytab_hbm.at[ch].at[sidx.at[b]], gbuf)
            pltpu.sync_copy(gbuf, accum.at[didx.at[b]], add=True)

        plsc.subcore_barrier()
        pltpu.sync_copy(accum.at[pl.ds(s * ROWS_PER_SUB, ROWS_PER_SUB)],
                        agg_hbm.at[ch].at[pl.ds(s * ROWS_PER_SUB, ROWS_PER_SUB)])
        plsc.subcore_barrier()


def _sc_agg(ytab, src2d, dst2d, zrows):
    nch = ytab.shape[0]
    return pl.kernel(
        functools.partial(_sc_agg_body, nch),
        out_type=jax.ShapeDtypeStruct((nch, NP, FC), jnp.float32),
        mesh=_mesh,
        scratch_types=[
            pltpu.VMEM((AGG_NB, BATCH), jnp.int32),
            pltpu.VMEM((AGG_NB, BATCH), jnp.int32),
            pltpu.VMEM((BATCH, FC), jnp.float32),
            pltpu.VMEM_SHARED((NP, FC), jnp.float32),
        ],
    )(ytab, src2d, dst2d, zrows)


def _tc_dinv_body(degp_ref, mask_ref, o_ref):
    deg = jnp.sum(degp_ref[:, :, 0:1], axis=0) + 1.0
    o_ref[...] = lax.rsqrt(deg) * mask_ref[...]


def _tc_dinv(degp, rowmask):
    degp3 = degp.reshape(NCORES * NSUB, NP, HC)
    return pl.pallas_call(
        _tc_dinv_body,
        grid=(NP // BM,),
        in_specs=[
            pl.BlockSpec((NCORES * NSUB, BM, HC), lambda i: (0, i, 0)),
            pl.BlockSpec((BM, 1), lambda i: (i, 0)),
        ],
        out_specs=pl.BlockSpec((BM, 1), lambda i: (i, 0)),
        out_shape=jax.ShapeDtypeStruct((NP, 1), jnp.float32),
    )(degp3, rowmask)


def _tc_mm_body(dinv_ref, x_ref, w_ref, o_ref):
    acc = jnp.dot(x_ref[...], w_ref[...], preferred_element_type=jnp.float32)
    o_ref[0] = acc * dinv_ref[...]


def _tc_mm(dinv, xp, w):
    k_in, d_out = w.shape
    nch = d_out // FC
    return pl.pallas_call(
        _tc_mm_body,
        grid=(NP // BM, nch),
        in_specs=[
            pl.BlockSpec((BM, 1), lambda i, j: (i, 0)),
            pl.BlockSpec((BM, k_in), lambda i, j: (i, 0)),
            pl.BlockSpec((k_in, FC), lambda i, j: (0, j)),
        ],
        out_specs=pl.BlockSpec((1, BM, FC), lambda i, j: (j, i, 0)),
        out_shape=jax.ShapeDtypeStruct((nch, NP, FC), jnp.float32),
    )(dinv, xp, w)


def _tc_ew_body(bn, agg_ref, y_ref, dinv_ref, b_ref, g_ref, bt_ref, o_ref):
    v = (agg_ref[0] + y_ref[0]) * dinv_ref[...] + b_ref[...]
    if bn:
        inv_s = (1.0 + 1e-5) ** -0.5
        v = v * (g_ref[...] * inv_s) + bt_ref[...]
        v = jnp.maximum(v, 0.0)
    o_ref[...] = v


def _tc_ew(agg, y, dinv, b, gamma, beta, bn):
    nch = agg.shape[0]
    d_out = nch * FC
    b2 = b.reshape(1, d_out)
    g2 = (gamma if bn else b).reshape(1, d_out)
    bt2 = (beta if bn else b).reshape(1, d_out)
    return pl.pallas_call(
        functools.partial(_tc_ew_body, bn),
        grid=(NP // BM, nch),
        in_specs=[
            pl.BlockSpec((1, BM, FC), lambda i, j: (j, i, 0)),
            pl.BlockSpec((1, BM, FC), lambda i, j: (j, i, 0)),
            pl.BlockSpec((BM, 1), lambda i, j: (i, 0)),
            pl.BlockSpec((1, FC), lambda i, j: (0, j)),
            pl.BlockSpec((1, FC), lambda i, j: (0, j)),
            pl.BlockSpec((1, FC), lambda i, j: (0, j)),
        ],
        out_specs=pl.BlockSpec((BM, FC), lambda i, j: (i, j)),
        out_shape=jax.ShapeDtypeStruct((NP, d_out), jnp.float32),
    )(agg, y, dinv, b2, g2, bt2)


def kernel(x, edge_index, W1, b1, gamma1, beta1, W2, b2, gamma2, beta2,
           W3, b3):
    ei = jnp.asarray(edge_index, jnp.int32)
    src = jnp.concatenate(
        [ei[0], jnp.full((EP - E,), PAD_SRC, jnp.int32)]).reshape(EP // BATCH,
                                                                  BATCH)
    dst = jnp.concatenate(
        [ei[1], jnp.full((EP - E,), TRASH, jnp.int32)]).reshape(EP // BATCH,
                                                                BATCH)
    xp = jnp.pad(x, ((0, NP - N), (0, 0)))
    zrows = jnp.zeros((ROWS_PER_SUB, FC), jnp.float32)
    rowmask = (jnp.arange(NP) < N).astype(jnp.float32).reshape(NP, 1)

    degp = _sc_deg(dst)
    dinv = _tc_dinv(degp, rowmask)

    h = xp
    for (w, b, g, bt, bn) in (
        (W1, b1, gamma1, beta1, True),
        (W2, b2, gamma2, beta2, True),
        (W3, b3, None, None, False),
    ):
        y = _tc_mm(dinv, h, w)
        agg = _sc_agg(y, src, dst, zrows)
        h = _tc_ew(agg, y, dinv, b, g, bt, bn)
    return h[:N]


# speedup vs baseline: 1.0916x; 1.0916x over previous
import dataclasses
import functools

import jax
import jax.numpy as jnp
from jax import lax
from jax.experimental import pallas as pl
from jax.experimental.pallas import tpu as pltpu
from jax.experimental.pallas import tpu_sc as plsc

N = 10000
NP = 10240
E = 160000
EP = 163840
FC = 128
NCORES = 2
NSUB = 16
BATCH = 128
TRASH = NP - 8
PAD_SRC = N
BM = 1280
ROWS_PER_SUB = NP // NSUB
DEG_NB = EP // (NCORES * NSUB) // BATCH
AGG_NB = EP // NSUB // BATCH
AGG_NBH = AGG_NB // 2

_mesh = plsc.VectorSubcoreMesh(core_axis_name="c", subcore_axis_name="s")

_sc_cp = pltpu.CompilerParams()
if "needs_layout_passes" in pltpu.CompilerParams.__dataclass_fields__:
    _sc_cp = dataclasses.replace(_sc_cp, needs_layout_passes=False)


HC = 8


def _sc_deg_body(dst_hbm, degp_hbm, didx, hist):
    c = lax.axis_index("c")
    s = lax.axis_index("s")
    wid = c * NSUB + s
    pltpu.sync_copy(dst_hbm.at[pl.ds(wid * DEG_NB, DEG_NB)], didx)

    @pl.loop(0, NP * HC, step=16)
    def _(i):
        hist[pl.ds(i, 16)] = jnp.zeros((16,), jnp.float32)

    ones_v = jnp.ones((16,), jnp.float32)

    @pl.loop(0, DEG_NB)
    def _(b):
        @pl.loop(0, BATCH, step=16)
        def _(j):
            idx = didx[b, pl.ds(j, 16)] * HC
            plsc.addupdate_scatter(hist, [idx], ones_v)

    pltpu.sync_copy(hist, degp_hbm.at[wid])


def _sc_deg(dst2d):
    return pl.kernel(
        _sc_deg_body,
        out_type=jax.ShapeDtypeStruct((NCORES * NSUB, NP * HC), jnp.float32),
        mesh=_mesh,
        compiler_params=_sc_cp,
        scratch_types=[
            pltpu.VMEM((DEG_NB, BATCH), jnp.int32),
            pltpu.VMEM((NP * HC,), jnp.float32),
        ],
    )(dst2d)


def _sc_agg_body(nch, ytab_hbm, src_hbm, dst_hbm, zeros_hbm, agg_hbm,
                 sidx, didx, gb0, gb1, accum):
    pl.run_scoped(
        functools.partial(_sc_agg_inner, nch, ytab_hbm, src_hbm, dst_hbm,
                          zeros_hbm, agg_hbm, sidx, didx, gb0, gb1, accum),
        pltpu.SemaphoreType.DMA(()),
        pltpu.SemaphoreType.DMA(()),
        pltpu.SemaphoreType.DMA(()),
        pltpu.SemaphoreType.DMA(()),
    )


def _sc_agg_inner(nch, ytab_hbm, src_hbm, dst_hbm, zeros_hbm, agg_hbm,
                  sidx, didx, gb0, gb1, accum, gs0, gs1, ss0, ss1):
    c = lax.axis_index("c")
    s = lax.axis_index("s")

    def gather_start(ch, b, gb, gs):
        pltpu.make_async_copy(ytab_hbm.at[ch].at[sidx.at[b]], gb, gs).start()

    def gather_wait(ch, b, gb, gs):
        pltpu.make_async_copy(ytab_hbm.at[ch].at[sidx.at[b]], gb, gs).wait()

    def scat_start(b, gb, ss):
        pltpu.make_async_copy(gb, accum.at[didx.at[b]], ss).start(add=True)

    def scat_wait(b, gb, ss):
        pltpu.make_async_copy(gb, accum.at[didx.at[b]], ss).wait()

    for k in range(nch // NCORES):
        ch = c + NCORES * k
        pltpu.sync_copy(zeros_hbm,
                        accum.at[pl.ds(s * ROWS_PER_SUB, ROWS_PER_SUB)])
        plsc.subcore_barrier()

        for h in range(2):
            base = s * AGG_NB + h * AGG_NBH
            pltpu.sync_copy(src_hbm.at[pl.ds(base, AGG_NBH)], sidx)
            pltpu.sync_copy(dst_hbm.at[pl.ds(base, AGG_NBH)], didx)

            gather_start(ch, 0, gb0, gs0)
            gather_start(ch, 1, gb1, gs1)

            @pl.loop(0, AGG_NBH // 2 - 1)
            def _(i):
                b = 2 * i
                gather_wait(ch, b, gb0, gs0)
                scat_start(b, gb0, ss0)
                gather_wait(ch, b + 1, gb1, gs1)
                scat_start(b + 1, gb1, ss1)
                scat_wait(b, gb0, ss0)
                gather_start(ch, b + 2, gb0, gs0)
                scat_wait(b + 1, gb1, ss1)
                gather_start(ch, b + 3, gb1, gs1)

            gather_wait(ch, AGG_NBH - 2, gb0, gs0)
            scat_start(AGG_NBH - 2, gb0, ss0)
            gather_wait(ch, AGG_NBH - 1, gb1, gs1)
            scat_start(AGG_NBH - 1, gb1, ss1)
            scat_wait(AGG_NBH - 2, gb0, ss0)
            scat_wait(AGG_NBH - 1, gb1, ss1)

        plsc.subcore_barrier()
        pltpu.sync_copy(accum.at[pl.ds(s * ROWS_PER_SUB, ROWS_PER_SUB)],
                        agg_hbm.at[ch].at[pl.ds(s * ROWS_PER_SUB, ROWS_PER_SUB)])
        plsc.subcore_barrier()


def _sc_agg(ytab, src2d, dst2d, zrows):
    nch = ytab.shape[0]
    return pl.kernel(
        functools.partial(_sc_agg_body, nch),
        out_type=jax.ShapeDtypeStruct((nch, NP, FC), jnp.float32),
        mesh=_mesh,
        scratch_types=[
            pltpu.VMEM((AGG_NBH, BATCH), jnp.int32),
            pltpu.VMEM((AGG_NBH, BATCH), jnp.int32),
            pltpu.VMEM((BATCH, FC), jnp.float32),
            pltpu.VMEM((BATCH, FC), jnp.float32),
            pltpu.VMEM_SHARED((NP, FC), jnp.float32),
        ],
    )(ytab, src2d, dst2d, zrows)


def _tc_dinv_body(degp_ref, mask_ref, o_ref):
    deg = jnp.sum(degp_ref[:, :, 0:1], axis=0) + 1.0
    o_ref[...] = lax.rsqrt(deg) * mask_ref[...]


def _tc_dinv(degp, rowmask):
    degp3 = degp.reshape(NCORES * NSUB, NP, HC)
    return pl.pallas_call(
        _tc_dinv_body,
        grid=(NP // BM,),
        in_specs=[
            pl.BlockSpec((NCORES * NSUB, BM, HC), lambda i: (0, i, 0)),
            pl.BlockSpec((BM, 1), lambda i: (i, 0)),
        ],
        out_specs=pl.BlockSpec((BM, 1), lambda i: (i, 0)),
        out_shape=jax.ShapeDtypeStruct((NP, 1), jnp.float32),
    )(degp3, rowmask)


def _tc_mm_body(dinv_ref, x_ref, w_ref, o_ref):
    acc = jnp.dot(x_ref[...], w_ref[...], preferred_element_type=jnp.float32)
    o_ref[0] = acc * dinv_ref[...]


def _tc_mm(dinv, xp, w):
    k_in, d_out = w.shape
    nch = d_out // FC
    return pl.pallas_call(
        _tc_mm_body,
        grid=(NP // BM, nch),
        in_specs=[
            pl.BlockSpec((BM, 1), lambda i, j: (i, 0)),
            pl.BlockSpec((BM, k_in), lambda i, j: (i, 0)),
            pl.BlockSpec((k_in, FC), lambda i, j: (0, j)),
        ],
        out_specs=pl.BlockSpec((1, BM, FC), lambda i, j: (j, i, 0)),
        out_shape=jax.ShapeDtypeStruct((nch, NP, FC), jnp.float32),
    )(dinv, xp, w)


def _tc_ew_body(bn, agg_ref, y_ref, dinv_ref, b_ref, g_ref, bt_ref, o_ref):
    v = (agg_ref[0] + y_ref[0]) * dinv_ref[...] + b_ref[...]
    if bn:
        inv_s = (1.0 + 1e-5) ** -0.5
        v = v * (g_ref[...] * inv_s) + bt_ref[...]
        v = jnp.maximum(v, 0.0)
    o_ref[...] = v


def _tc_ew(agg, y, dinv, b, gamma, beta, bn):
    nch = agg.shape[0]
    d_out = nch * FC
    b2 = b.reshape(1, d_out)
    g2 = (gamma if bn else b).reshape(1, d_out)
    bt2 = (beta if bn else b).reshape(1, d_out)
    return pl.pallas_call(
        functools.partial(_tc_ew_body, bn),
        grid=(NP // BM, nch),
        in_specs=[
            pl.BlockSpec((1, BM, FC), lambda i, j: (j, i, 0)),
            pl.BlockSpec((1, BM, FC), lambda i, j: (j, i, 0)),
            pl.BlockSpec((BM, 1), lambda i, j: (i, 0)),
            pl.BlockSpec((1, FC), lambda i, j: (0, j)),
            pl.BlockSpec((1, FC), lambda i, j: (0, j)),
            pl.BlockSpec((1, FC), lambda i, j: (0, j)),
        ],
        out_specs=pl.BlockSpec((BM, FC), lambda i, j: (i, j)),
        out_shape=jax.ShapeDtypeStruct((NP, d_out), jnp.float32),
    )(agg, y, dinv, b2, g2, bt2)


def kernel(x, edge_index, W1, b1, gamma1, beta1, W2, b2, gamma2, beta2,
           W3, b3):
    ei = jnp.asarray(edge_index, jnp.int32)
    src = jnp.concatenate(
        [ei[0], jnp.full((EP - E,), PAD_SRC, jnp.int32)]).reshape(EP // BATCH,
                                                                  BATCH)
    dst = jnp.concatenate(
        [ei[1], jnp.full((EP - E,), TRASH, jnp.int32)]).reshape(EP // BATCH,
                                                                BATCH)
    xp = jnp.pad(x, ((0, NP - N), (0, 0)))
    zrows = jnp.zeros((ROWS_PER_SUB, FC), jnp.float32)
    rowmask = (jnp.arange(NP) < N).astype(jnp.float32).reshape(NP, 1)

    degp = _sc_deg(dst)
    dinv = _tc_dinv(degp, rowmask)

    h = xp
    for (w, b, g, bt, bn) in (
        (W1, b1, gamma1, beta1, True),
        (W2, b2, gamma2, beta2, True),
        (W3, b3, None, None, False),
    ):
        y = _tc_mm(dinv, h, w)
        agg = _sc_agg(y, src, dst, zrows)
        h = _tc_ew(agg, y, dinv, b, g, bt, bn)
    return h[:N]


# speedup vs baseline: 1.1327x; 1.0377x over previous
import dataclasses
import functools

import jax
import jax.numpy as jnp
from jax import lax
from jax.experimental import pallas as pl
from jax.experimental.pallas import tpu as pltpu
from jax.experimental.pallas import tpu_sc as plsc

N = 10000
NP = 10240
E = 160000
EP = 163840
FC = 128
NCORES = 2
NSUB = 16
BATCH = 128
TRASH = NP - 8
PAD_SRC = N
BM = 1280
ROWS_PER_SUB = NP // NSUB
DEG_NB = EP // (NCORES * NSUB) // BATCH
AGG_NB = EP // NSUB // BATCH
AGG_NBH = AGG_NB // 2

_mesh = plsc.VectorSubcoreMesh(core_axis_name="c", subcore_axis_name="s")

_sc_cp = pltpu.CompilerParams()
if "needs_layout_passes" in pltpu.CompilerParams.__dataclass_fields__:
    _sc_cp = dataclasses.replace(_sc_cp, needs_layout_passes=False)


HC = 2


def _sc_deg_body(dst_hbm, degp_hbm, didx, hist):
    c = lax.axis_index("c")
    s = lax.axis_index("s")
    wid = c * NSUB + s
    pltpu.sync_copy(dst_hbm.at[pl.ds(wid * DEG_NB, DEG_NB)], didx)

    @pl.loop(0, NP * HC, step=16)
    def _(i):
        hist[pl.ds(i, 16)] = jnp.zeros((16,), jnp.float32)

    ones_v = jnp.ones((16,), jnp.float32)

    @pl.loop(0, DEG_NB)
    def _(b):
        @pl.loop(0, BATCH, step=16)
        def _(j):
            idx = didx[b, pl.ds(j, 16)] * HC
            plsc.addupdate_scatter(hist, [idx], ones_v)

    pltpu.sync_copy(hist, degp_hbm.at[wid])


def _sc_deg(dst2d):
    return pl.kernel(
        _sc_deg_body,
        out_type=jax.ShapeDtypeStruct((NCORES * NSUB, NP * HC), jnp.float32),
        mesh=_mesh,
        compiler_params=_sc_cp,
        scratch_types=[
            pltpu.VMEM((DEG_NB, BATCH), jnp.int32),
            pltpu.VMEM((NP * HC,), jnp.float32),
        ],
    )(dst2d)


def _sc_agg_body(nch, ytab_hbm, src_hbm, dst_hbm, zeros_hbm, agg_hbm,
                 sidx, didx, gb0, gb1, accum):
    pl.run_scoped(
        functools.partial(_sc_agg_inner, nch, ytab_hbm, src_hbm, dst_hbm,
                          zeros_hbm, agg_hbm, sidx, didx, gb0, gb1, accum),
        pltpu.SemaphoreType.DMA(()),
        pltpu.SemaphoreType.DMA(()),
        pltpu.SemaphoreType.DMA(()),
        pltpu.SemaphoreType.DMA(()),
    )


def _sc_agg_inner(nch, ytab_hbm, src_hbm, dst_hbm, zeros_hbm, agg_hbm,
                  sidx, didx, gb0, gb1, accum, gs0, gs1, ss0, ss1):
    c = lax.axis_index("c")
    s = lax.axis_index("s")

    def gather_start(ch, b, gb, gs):
        pltpu.make_async_copy(ytab_hbm.at[ch].at[sidx.at[b]], gb, gs).start()

    def gather_wait(ch, b, gb, gs):
        pltpu.make_async_copy(ytab_hbm.at[ch].at[sidx.at[b]], gb, gs).wait()

    def scat_start(b, gb, ss):
        pltpu.make_async_copy(gb, accum.at[didx.at[b]], ss).start(add=True)

    def scat_wait(b, gb, ss):
        pltpu.make_async_copy(gb, accum.at[didx.at[b]], ss).wait()

    for k in range(nch // NCORES):
        ch = c + NCORES * k
        pltpu.sync_copy(zeros_hbm,
                        accum.at[pl.ds(s * ROWS_PER_SUB, ROWS_PER_SUB)])
        plsc.subcore_barrier()

        for h in range(2):
            base = s * AGG_NB + h * AGG_NBH
            pltpu.sync_copy(src_hbm.at[pl.ds(base, AGG_NBH)], sidx)
            pltpu.sync_copy(dst_hbm.at[pl.ds(base, AGG_NBH)], didx)

            gather_start(ch, 0, gb0, gs0)
            gather_start(ch, 1, gb1, gs1)

            @pl.loop(0, AGG_NBH // 2 - 1)
            def _(i):
                b = 2 * i
                gather_wait(ch, b, gb0, gs0)
                scat_start(b, gb0, ss0)
                gather_wait(ch, b + 1, gb1, gs1)
                scat_start(b + 1, gb1, ss1)
                scat_wait(b, gb0, ss0)
                gather_start(ch, b + 2, gb0, gs0)
                scat_wait(b + 1, gb1, ss1)
                gather_start(ch, b + 3, gb1, gs1)

            gather_wait(ch, AGG_NBH - 2, gb0, gs0)
            scat_start(AGG_NBH - 2, gb0, ss0)
            gather_wait(ch, AGG_NBH - 1, gb1, gs1)
            scat_start(AGG_NBH - 1, gb1, ss1)
            scat_wait(AGG_NBH - 2, gb0, ss0)
            scat_wait(AGG_NBH - 1, gb1, ss1)

        plsc.subcore_barrier()
        pltpu.sync_copy(accum.at[pl.ds(s * ROWS_PER_SUB, ROWS_PER_SUB)],
                        agg_hbm.at[ch].at[pl.ds(s * ROWS_PER_SUB, ROWS_PER_SUB)])
        plsc.subcore_barrier()


def _sc_agg(ytab, src2d, dst2d, zrows):
    nch = ytab.shape[0]
    return pl.kernel(
        functools.partial(_sc_agg_body, nch),
        out_type=jax.ShapeDtypeStruct((nch, NP, FC), jnp.float32),
        mesh=_mesh,
        scratch_types=[
            pltpu.VMEM((AGG_NBH, BATCH), jnp.int32),
            pltpu.VMEM((AGG_NBH, BATCH), jnp.int32),
            pltpu.VMEM((BATCH, FC), jnp.float32),
            pltpu.VMEM((BATCH, FC), jnp.float32),
            pltpu.VMEM_SHARED((NP, FC), jnp.float32),
        ],
    )(ytab, src2d, dst2d, zrows)


def _tc_dinv_body(degp_ref, mask_ref, o_ref):
    deg = jnp.sum(degp_ref[:, :, 0:1], axis=0) + 1.0
    o_ref[...] = lax.rsqrt(deg) * mask_ref[...]


def _tc_dinv(degp, rowmask):
    degp3 = degp.reshape(NCORES * NSUB, NP, HC)
    return pl.pallas_call(
        _tc_dinv_body,
        grid=(NP // BM,),
        in_specs=[
            pl.BlockSpec((NCORES * NSUB, BM, HC), lambda i: (0, i, 0)),
            pl.BlockSpec((BM, 1), lambda i: (i, 0)),
        ],
        out_specs=pl.BlockSpec((BM, 1), lambda i: (i, 0)),
        out_shape=jax.ShapeDtypeStruct((NP, 1), jnp.float32),
    )(degp3, rowmask)


def _tc_mm_body(dinv_ref, x_ref, w_ref, o_ref):
    acc = jnp.dot(x_ref[...], w_ref[...], preferred_element_type=jnp.float32)
    o_ref[0] = acc * dinv_ref[...]


def _tc_mm(dinv, xp, w):
    k_in, d_out = w.shape
    nch = d_out // FC
    return pl.pallas_call(
        _tc_mm_body,
        grid=(NP // BM, nch),
        in_specs=[
            pl.BlockSpec((BM, 1), lambda i, j: (i, 0)),
            pl.BlockSpec((BM, k_in), lambda i, j: (i, 0)),
            pl.BlockSpec((k_in, FC), lambda i, j: (0, j)),
        ],
        out_specs=pl.BlockSpec((1, BM, FC), lambda i, j: (j, i, 0)),
        out_shape=jax.ShapeDtypeStruct((nch, NP, FC), jnp.float32),
    )(dinv, xp, w)


def _tc_ew_body(bn, agg_ref, y_ref, dinv_ref, b_ref, g_ref, bt_ref, o_ref):
    v = (agg_ref[0] + y_ref[0]) * dinv_ref[...] + b_ref[...]
    if bn:
        inv_s = (1.0 + 1e-5) ** -0.5
        v = v * (g_ref[...] * inv_s) + bt_ref[...]
        v = jnp.maximum(v, 0.0)
    o_ref[...] = v


def _tc_ew(agg, y, dinv, b, gamma, beta, bn):
    nch = agg.shape[0]
    d_out = nch * FC
    b2 = b.reshape(1, d_out)
    g2 = (gamma if bn else b).reshape(1, d_out)
    bt2 = (beta if bn else b).reshape(1, d_out)
    return pl.pallas_call(
        functools.partial(_tc_ew_body, bn),
        grid=(NP // BM, nch),
        in_specs=[
            pl.BlockSpec((1, BM, FC), lambda i, j: (j, i, 0)),
            pl.BlockSpec((1, BM, FC), lambda i, j: (j, i, 0)),
            pl.BlockSpec((BM, 1), lambda i, j: (i, 0)),
            pl.BlockSpec((1, FC), lambda i, j: (0, j)),
            pl.BlockSpec((1, FC), lambda i, j: (0, j)),
            pl.BlockSpec((1, FC), lambda i, j: (0, j)),
        ],
        out_specs=pl.BlockSpec((BM, FC), lambda i, j: (i, j)),
        out_shape=jax.ShapeDtypeStruct((NP, d_out), jnp.float32),
    )(agg, y, dinv, b2, g2, bt2)


def kernel(x, edge_index, W1, b1, gamma1, beta1, W2, b2, gamma2, beta2,
           W3, b3):
    ei = jnp.asarray(edge_index, jnp.int32)
    src = jnp.concatenate(
        [ei[0], jnp.full((EP - E,), PAD_SRC, jnp.int32)]).reshape(EP // BATCH,
                                                                  BATCH)
    dst = jnp.concatenate(
        [ei[1], jnp.full((EP - E,), TRASH, jnp.int32)]).reshape(EP // BATCH,
                                                                BATCH)
    xp = jnp.pad(x, ((0, NP - N), (0, 0)))
    zrows = jnp.zeros((ROWS_PER_SUB, FC), jnp.float32)
    rowmask = (jnp.arange(NP) < N).astype(jnp.float32).reshape(NP, 1)

    degp = _sc_deg(dst)
    dinv = _tc_dinv(degp, rowmask)

    h = xp
    for (w, b, g, bt, bn) in (
        (W1, b1, gamma1, beta1, True),
        (W2, b2, gamma2, beta2, True),
        (W3, b3, None, None, False),
    ):
        y = _tc_mm(dinv, h, w)
        agg = _sc_agg(y, src, dst, zrows)
        h = _tc_ew(agg, y, dinv, b, g, bt, bn)
    return h[:N]
